# baseline (speedup 1.0000x reference)
import functools

import numpy as np

import jax
import jax.numpy as jnp
from jax import lax
from jax.experimental import pallas as pl
from jax.experimental.pallas import tpu as pltpu
from jax.experimental.pallas import tpu_sc as plsc

N_ATOMS = 110000
D = 128
PER_DEG = 10000
MAX_DEG = 10

NC = 2
NS = 16
NW = NC * NS

CHUNK = 320
NLANE = 16
NSLOT = D // NLANE

DEGS_A = (1, 2, 3, 4, 5, 6, 7, 8)
DEGS_B = (9, 10)


def _make_sc_body(degs):
    n_cols = sum(degs)
    col_of = {}
    run = 0
    for d in degs:
        col_of[d] = run
        run += d

    def body(table, *rest):
        idxs = dict(zip(degs, rest[:len(degs)]))
        out = rest[len(degs)]
        idx_v = rest[len(degs) + 1:len(degs) + 1 + n_cols]
        acc_v, pk_v, sem_idx, sem_add0, sem_add1 = rest[len(degs) + 1 + n_cols:]
        sem_add = (sem_add0, sem_add1)

        w = lax.axis_index("s") * NC + lax.axis_index("c")
        base = pl.multiple_of(
            jnp.where(w == NW - 1, PER_DEG - CHUNK, w * CHUNK), 8)

        idx_cps = []
        for d in degs:
            for j in range(d):
                idx_cps.append(pltpu.async_copy(
                    idxs[d].at[pl.ds(pl.multiple_of(j * PER_DEG + base, 8), CHUNK)],
                    idx_v[col_of[d] + j],
                    sem_idx,
                ))

        def zero_acc(p):
            zeros = jnp.zeros((NLANE,), jnp.float32)

            def zrow(i, _):
                for s in range(NSLOT):
                    acc_v[p, i, pl.ds(s * NLANE, NLANE)] = zeros
                return 0

            lax.fori_loop(0, CHUNK, zrow, 0)

        def fire_adds(p, d):
            return [
                pltpu.async_copy(
                    table.at[idx_v[col_of[d] + j]], acc_v.at[p], sem_add[p],
                    add=True)
                for j in range(d)
            ]

        def store(p, q):
            H = CHUNK // 2
            for h in range(2):
                def prow(i, _, h=h):
                    for t in range(NSLOT // 2):
                        a = acc_v[p, h * H + i, pl.ds(2 * t * NLANE, NLANE)]
                        bb = acc_v[p, h * H + i, pl.ds((2 * t + 1) * NLANE, NLANE)]
                        v = plsc.bitcast(
                            plsc.pack(a, bb, format=plsc.PackFormat.INTERLEAVED),
                            jnp.int32)
                        pk_v[i, pl.ds(t * NLANE, NLANE)] = v
                    return 0

                lax.fori_loop(0, H, prow, 0)
                dst = pl.multiple_of(q * PER_DEG + base + h * H, 8)
                pltpu.sync_copy(pk_v, out.at[pl.ds(dst, H)])

        for cp in idx_cps:
            cp.wait()
        pending = [None, None]
        for q in range(min(2, len(degs))):
            zero_acc(q)
            pending[q] = fire_adds(q, degs[q])
        for q in range(2, len(degs)):
            p = q % 2
            for cp in pending[p]:
                cp.wait()
            store(p, q - 2)
            zero_acc(p)
            pending[p] = fire_adds(p, degs[q])
        for q in range(max(0, len(degs) - 2), len(degs)):
            p = q % 2
            for cp in pending[p]:
                cp.wait()
            store(p, q)

    return body, n_cols


def _sc_gather_sum(atom_features, idx_cols, degs):
    body, n_cols = _make_sc_body(degs)
    mesh = plsc.VectorSubcoreMesh(
        core_axis_name="c", subcore_axis_name="s", num_cores=NC, num_subcores=NS
    )
    fn = pl.kernel(
        body,
        out_type=jax.ShapeDtypeStruct((len(degs) * PER_DEG, D // 2), jnp.int32),
        mesh=mesh,
        compiler_params=pltpu.CompilerParams(needs_layout_passes=False),
        scratch_types=(
            [pltpu.VMEM((CHUNK,), jnp.int32)] * n_cols
            + [
                pltpu.VMEM((2, CHUNK, D), jnp.float32),
                pltpu.VMEM((CHUNK // 2, D // 2), jnp.int32),
                pltpu.SemaphoreType.DMA,
                pltpu.SemaphoreType.DMA,
                pltpu.SemaphoreType.DMA,
            ]
        ),
    )
    return fn(atom_features, *[idx_cols[d - 1] for d in degs])


ROWS_PER_TILE = PER_DEG


def _tc_body(atom_ref, summed_ref, ws_ref, wr_ref, bs_ref, br_ref, out_ref,
             *, first_bucket):
    bucket = pl.program_id(0) + first_bucket
    acc = jnp.dot(atom_ref[...], ws_ref[0], preferred_element_type=jnp.float32)
    rel = jnp.dot(summed_ref[...].astype(jnp.float32), wr_ref[0],
                  preferred_element_type=jnp.float32)
    rel = jnp.where(bucket == 0, 0.0, rel + br_ref[0])
    out_ref[...] = acc + rel + bs_ref[0]


def _tc_body_b(atom_ref, summed_ref, ws_ref, wr_ref, bs_ref, br_ref, alias_ref,
               out_ref):
    _tc_body(atom_ref, summed_ref, ws_ref, wr_ref, bs_ref, br_ref, out_ref,
             first_bucket=9)


def _weight_maps(first_bucket):
    def self_idx(i):
        bkt = i + first_bucket
        return jnp.where(bkt == 0, 2 * MAX_DEG, 2 * bkt - 1)

    def rel_idx(i):
        bkt = i + first_bucket
        return jnp.where(bkt == 0, 0, 2 * bkt - 2)

    return self_idx, rel_idx


PACK_PERM = np.array(
    [32 * (p // 32) + (p % 32) // 2 + 16 * (p % 2) for p in range(D)],
    dtype=np.int32)


def _as_bf16(summed_i32):
    v = jax.lax.bitcast_convert_type(summed_i32, jnp.bfloat16)
    return v.reshape(summed_i32.shape[0], D)


def _tc_matmul_a(atom_features, summed_a, W, Wp, b3):
    self_idx, rel_idx = _weight_maps(0)
    return pl.pallas_call(
        functools.partial(_tc_body, first_bucket=0),
        grid=(9,),
        in_specs=[
            pl.BlockSpec((ROWS_PER_TILE, D), lambda i: (i, 0)),
            pl.BlockSpec((ROWS_PER_TILE, D), lambda i: (jnp.maximum(i - 1, 0), 0)),
            pl.BlockSpec((1, D, D), lambda i: (self_idx(i), 0, 0)),
            pl.BlockSpec((1, D, D), lambda i: (rel_idx(i), 0, 0)),
            pl.BlockSpec((1, 1, D), lambda i: (self_idx(i), 0, 0)),
            pl.BlockSpec((1, 1, D), lambda i: (rel_idx(i), 0, 0)),
        ],
        out_specs=pl.BlockSpec((ROWS_PER_TILE, D), lambda i: (i, 0)),
        out_shape=jax.ShapeDtypeStruct((N_ATOMS, D), jnp.float32),
    )(atom_features, summed_a, W, Wp, b3, b3)


def _tc_matmul_b(partial_out, atom_features, summed_b, W, Wp, b3):
    self_idx, rel_idx = _weight_maps(9)
    return pl.pallas_call(
        _tc_body_b,
        grid=(2,),
        in_specs=[
            pl.BlockSpec((ROWS_PER_TILE, D), lambda i: (i + 9, 0)),
            pl.BlockSpec((ROWS_PER_TILE, D), lambda i: (i, 0)),
            pl.BlockSpec((1, D, D), lambda i: (self_idx(i), 0, 0)),
            pl.BlockSpec((1, D, D), lambda i: (rel_idx(i), 0, 0)),
            pl.BlockSpec((1, 1, D), lambda i: (self_idx(i), 0, 0)),
            pl.BlockSpec((1, 1, D), lambda i: (rel_idx(i), 0, 0)),
            pl.BlockSpec(memory_space=pl.ANY),
        ],
        out_specs=pl.BlockSpec((ROWS_PER_TILE, D), lambda i: (i + 9, 0)),
        out_shape=jax.ShapeDtypeStruct((N_ATOMS, D), jnp.float32),
        input_output_aliases={6: 0},
    )(atom_features, summed_b, W, Wp, b3, b3, partial_out)


def kernel(atom_features, deg_slice, membership, deg_adj_1, deg_adj_2,
           deg_adj_3, deg_adj_4, deg_adj_5, deg_adj_6, deg_adj_7, deg_adj_8,
           deg_adj_9, deg_adj_10, W, b):
    adjs = [deg_adj_1, deg_adj_2, deg_adj_3, deg_adj_4, deg_adj_5,
            deg_adj_6, deg_adj_7, deg_adj_8, deg_adj_9, deg_adj_10]
    idx_cols = [a.T.reshape(-1) for a in adjs]
    summed_a = _as_bf16(_sc_gather_sum(atom_features, idx_cols, DEGS_A))
    summed_b = _as_bf16(_sc_gather_sum(atom_features, idx_cols, DEGS_B))
    b3 = b.reshape(2 * MAX_DEG + 1, 1, D)
    Wp = W[:, PACK_PERM, :]
    out_a = _tc_matmul_a(atom_features, summed_a, W, Wp, b3)
    return _tc_matmul_b(out_a, atom_features, summed_b, W, Wp, b3)


# speedup vs baseline: 2.2097x; 2.2097x over previous
import functools

import jax
import jax.numpy as jnp
from jax import lax
from jax.experimental import pallas as pl
from jax.experimental.pallas import tpu as pltpu
from jax.experimental.pallas import tpu_sc as plsc

N_ATOMS = 110000
D = 128
PER_DEG = 10000
MAX_DEG = 10

NC = 2
NS = 16
NW = NC * NS

CHUNK = 320
NLANE = 16
NSLOT = D // NLANE

DEGS_A = (1, 2, 3, 4, 5, 6, 7)
DEGS_B = (8, 9, 10)


def _make_sc_body(degs):
    n_cols = sum(degs)
    col_of = {}
    run = 0
    for d in degs:
        col_of[d] = run
        run += d

    def body(table, *rest):
        idxs = dict(zip(degs, rest[:len(degs)]))
        out = rest[len(degs)]
        idx_v = rest[len(degs) + 1:len(degs) + 1 + n_cols]
        acc_v, sem_idx, sem_add0, sem_add1 = rest[len(degs) + 1 + n_cols:]
        sem_add = (sem_add0, sem_add1)

        w = lax.axis_index("s") * NC + lax.axis_index("c")
        base = pl.multiple_of(
            jnp.where(w == NW - 1, PER_DEG - CHUNK, w * CHUNK), 8)

        idx_cps = []
        for d in degs:
            for j in range(d):
                idx_cps.append(pltpu.async_copy(
                    idxs[d].at[pl.ds(pl.multiple_of(j * PER_DEG + base, 8), CHUNK)],
                    idx_v[col_of[d] + j],
                    sem_idx,
                ))

        def zero_acc(p):
            zeros = jnp.zeros((NLANE,), jnp.float32)

            def zrow(i, _):
                for s in range(NSLOT):
                    acc_v[p, i, pl.ds(s * NLANE, NLANE)] = zeros
                return 0

            lax.fori_loop(0, CHUNK, zrow, 0)

        def fire_adds(p, d):
            return [
                pltpu.async_copy(
                    table.at[idx_v[col_of[d] + j]], acc_v.at[p], sem_add[p],
                    add=True)
                for j in range(d)
            ]

        def store(p, q):
            dst = pl.multiple_of(q * PER_DEG + base, 8)
            pltpu.sync_copy(acc_v.at[p], out.at[pl.ds(dst, CHUNK)])

        for cp in idx_cps:
            cp.wait()
        pending = [None, None]
        for q in range(min(2, len(degs))):
            zero_acc(q)
            pending[q] = fire_adds(q, degs[q])
        for q in range(2, len(degs)):
            p = q % 2
            for cp in pending[p]:
                cp.wait()
            store(p, q - 2)
            zero_acc(p)
            pending[p] = fire_adds(p, degs[q])
        for q in range(max(0, len(degs) - 2), len(degs)):
            p = q % 2
            for cp in pending[p]:
                cp.wait()
            store(p, q)

    return body, n_cols


def _sc_gather_sum(atom_features, idx_cols, degs):
    body, n_cols = _make_sc_body(degs)
    mesh = plsc.VectorSubcoreMesh(
        core_axis_name="c", subcore_axis_name="s", num_cores=NC, num_subcores=NS
    )
    fn = pl.kernel(
        body,
        out_type=jax.ShapeDtypeStruct((len(degs) * PER_DEG, D), jnp.float32),
        mesh=mesh,
        scratch_types=(
            [pltpu.VMEM((CHUNK,), jnp.int32)] * n_cols
            + [
                pltpu.VMEM((2, CHUNK, D), jnp.float32),
                pltpu.SemaphoreType.DMA,
                pltpu.SemaphoreType.DMA,
                pltpu.SemaphoreType.DMA,
            ]
        ),
    )
    return fn(atom_features, *[idx_cols[d - 1] for d in degs])


ROWS_PER_TILE = PER_DEG


def _tc_body(atom_ref, summed_ref, ws_ref, wr_ref, bs_ref, br_ref, out_ref,
             *, first_bucket):
    bucket = pl.program_id(0) + first_bucket
    acc = jnp.dot(atom_ref[...], ws_ref[0], preferred_element_type=jnp.float32)
    rel = jnp.dot(summed_ref[...], wr_ref[0], preferred_element_type=jnp.float32)
    rel = jnp.where(bucket == 0, 0.0, rel + br_ref[0])
    out_ref[...] = acc + rel + bs_ref[0]


def _tc_body_b(atom_ref, summed_ref, ws_ref, wr_ref, bs_ref, br_ref, alias_ref,
               out_ref):
    _tc_body(atom_ref, summed_ref, ws_ref, wr_ref, bs_ref, br_ref, out_ref,
             first_bucket=8)


def _weight_maps(first_bucket):
    def self_idx(i):
        bkt = i + first_bucket
        return jnp.where(bkt == 0, 2 * MAX_DEG, 2 * bkt - 1)

    def rel_idx(i):
        bkt = i + first_bucket
        return jnp.where(bkt == 0, 0, 2 * bkt - 2)

    return self_idx, rel_idx


def _tc_matmul_a(atom_features, summed_a, W, b3):
    self_idx, rel_idx = _weight_maps(0)
    return pl.pallas_call(
        functools.partial(_tc_body, first_bucket=0),
        grid=(8,),
        in_specs=[
            pl.BlockSpec((ROWS_PER_TILE, D), lambda i: (i, 0)),
            pl.BlockSpec((ROWS_PER_TILE, D), lambda i: (jnp.maximum(i - 1, 0), 0)),
            pl.BlockSpec((1, D, D), lambda i: (self_idx(i), 0, 0)),
            pl.BlockSpec((1, D, D), lambda i: (rel_idx(i), 0, 0)),
            pl.BlockSpec((1, 1, D), lambda i: (self_idx(i), 0, 0)),
            pl.BlockSpec((1, 1, D), lambda i: (rel_idx(i), 0, 0)),
        ],
        out_specs=pl.BlockSpec((ROWS_PER_TILE, D), lambda i: (i, 0)),
        out_shape=jax.ShapeDtypeStruct((N_ATOMS, D), jnp.float32),
    )(atom_features, summed_a, W, W, b3, b3)


def _tc_matmul_b(partial_out, atom_features, summed_b, W, b3):
    self_idx, rel_idx = _weight_maps(8)
    return pl.pallas_call(
        _tc_body_b,
        grid=(3,),
        in_specs=[
            pl.BlockSpec((ROWS_PER_TILE, D), lambda i: (i + 8, 0)),
            pl.BlockSpec((ROWS_PER_TILE, D), lambda i: (i, 0)),
            pl.BlockSpec((1, D, D), lambda i: (self_idx(i), 0, 0)),
            pl.BlockSpec((1, D, D), lambda i: (rel_idx(i), 0, 0)),
            pl.BlockSpec((1, 1, D), lambda i: (self_idx(i), 0, 0)),
            pl.BlockSpec((1, 1, D), lambda i: (rel_idx(i), 0, 0)),
            pl.BlockSpec(memory_space=pl.ANY),
        ],
        out_specs=pl.BlockSpec((ROWS_PER_TILE, D), lambda i: (i + 8, 0)),
        out_shape=jax.ShapeDtypeStruct((N_ATOMS, D), jnp.float32),
        input_output_aliases={6: 0},
    )(atom_features, summed_b, W, W, b3, b3, partial_out)


def kernel(atom_features, deg_slice, membership, deg_adj_1, deg_adj_2,
           deg_adj_3, deg_adj_4, deg_adj_5, deg_adj_6, deg_adj_7, deg_adj_8,
           deg_adj_9, deg_adj_10, W, b):
    adjs = [deg_adj_1, deg_adj_2, deg_adj_3, deg_adj_4, deg_adj_5,
            deg_adj_6, deg_adj_7, deg_adj_8, deg_adj_9, deg_adj_10]
    idx_cols = [a.T.reshape(-1) for a in adjs]
    summed_a = _sc_gather_sum(atom_features, idx_cols, DEGS_A)
    summed_b = _sc_gather_sum(atom_features, idx_cols, DEGS_B)
    b3 = b.reshape(2 * MAX_DEG + 1, 1, D)
    out_a = _tc_matmul_a(atom_features, summed_a, W, b3)
    return _tc_matmul_b(out_a, atom_features, summed_b, W, b3)


# speedup vs baseline: 2.2531x; 1.0196x over previous
import functools

import jax
import jax.numpy as jnp
from jax import lax
from jax.experimental import pallas as pl
from jax.experimental.pallas import tpu as pltpu
from jax.experimental.pallas import tpu_sc as plsc

N_ATOMS = 110000
D = 128
PER_DEG = 10000
MAX_DEG = 10

NC = 2
NS = 16
NW = NC * NS

CHUNK = 320
NLANE = 16
NSLOT = D // NLANE

STAGES = ((1, 2, 3), (4, 5, 6, 7), (8, 9, 10))


def _make_sc_body(degs):
    n_cols = sum(degs)
    col_of = {}
    run = 0
    for d in degs:
        col_of[d] = run
        run += d

    def body(table, *rest):
        idxs = dict(zip(degs, rest[:len(degs)]))
        out = rest[len(degs)]
        idx_v = rest[len(degs) + 1:len(degs) + 1 + n_cols]
        acc_v, sem_idx, sem_add0, sem_add1 = rest[len(degs) + 1 + n_cols:]
        sem_add = (sem_add0, sem_add1)

        w = lax.axis_index("s") * NC + lax.axis_index("c")
        base = pl.multiple_of(
            jnp.where(w == NW - 1, PER_DEG - CHUNK, w * CHUNK), 8)

        idx_cps = []
        for d in degs:
            for j in range(d):
                idx_cps.append(pltpu.async_copy(
                    idxs[d].at[pl.ds(pl.multiple_of(j * PER_DEG + base, 8), CHUNK)],
                    idx_v[col_of[d] + j],
                    sem_idx,
                ))

        def zero_acc(p):
            zeros = jnp.zeros((NLANE,), jnp.float32)

            def zrow(i, _):
                for s in range(NSLOT):
                    acc_v[p, i, pl.ds(s * NLANE, NLANE)] = zeros
                return 0

            lax.fori_loop(0, CHUNK, zrow, 0)

        def fire_adds(p, d):
            return [
                pltpu.async_copy(
                    table.at[idx_v[col_of[d] + j]], acc_v.at[p], sem_add[p],
                    add=True)
                for j in range(d)
            ]

        def store(p, q):
            dst = pl.multiple_of(q * PER_DEG + base, 8)
            pltpu.sync_copy(acc_v.at[p], out.at[pl.ds(dst, CHUNK)])

        for cp in idx_cps:
            cp.wait()
        pending = [None, None]
        for q in range(min(2, len(degs))):
            zero_acc(q)
            pending[q] = fire_adds(q, degs[q])
        for q in range(2, len(degs)):
            p = q % 2
            for cp in pending[p]:
                cp.wait()
            store(p, q - 2)
            zero_acc(p)
            pending[p] = fire_adds(p, degs[q])
        for q in range(max(0, len(degs) - 2), len(degs)):
            p = q % 2
            for cp in pending[p]:
                cp.wait()
            store(p, q)

    return body, n_cols


def _sc_gather_sum(atom_features, idx_cols, degs):
    body, n_cols = _make_sc_body(degs)
    mesh = plsc.VectorSubcoreMesh(
        core_axis_name="c", subcore_axis_name="s", num_cores=NC, num_subcores=NS
    )
    fn = pl.kernel(
        body,
        out_type=jax.ShapeDtypeStruct((len(degs) * PER_DEG, D), jnp.float32),
        mesh=mesh,
        scratch_types=(
            [pltpu.VMEM((CHUNK,), jnp.int32)] * n_cols
            + [
                pltpu.VMEM((2, CHUNK, D), jnp.float32),
                pltpu.SemaphoreType.DMA,
                pltpu.SemaphoreType.DMA,
                pltpu.SemaphoreType.DMA,
            ]
        ),
    )
    return fn(atom_features, *[idx_cols[d - 1] for d in degs])


ROWS_PER_TILE = PER_DEG


def _tc_body(atom_ref, summed_ref, ws_ref, wr_ref, bs_ref, br_ref, out_ref,
             *, first_bucket):
    bucket = pl.program_id(0) + first_bucket
    acc = jnp.dot(atom_ref[...], ws_ref[0], preferred_element_type=jnp.float32)
    rel = jnp.dot(summed_ref[...], wr_ref[0], preferred_element_type=jnp.float32)
    rel = jnp.where(bucket == 0, 0.0, rel + br_ref[0])
    out_ref[...] = acc + rel + bs_ref[0]


def _tc_body_alias(atom_ref, summed_ref, ws_ref, wr_ref, bs_ref, br_ref,
                   alias_ref, out_ref, *, first_bucket):
    _tc_body(atom_ref, summed_ref, ws_ref, wr_ref, bs_ref, br_ref, out_ref,
             first_bucket=first_bucket)


def _weight_maps(first_bucket):
    def self_idx(i):
        bkt = i + first_bucket
        return jnp.where(bkt == 0, 2 * MAX_DEG, 2 * bkt - 1)

    def rel_idx(i):
        bkt = i + first_bucket
        return jnp.where(bkt == 0, 0, 2 * bkt - 2)

    return self_idx, rel_idx


def _tc_matmul(atom_features, summed, W, b3, first_bucket, n_buckets,
               partial_out=None):
    self_idx, rel_idx = _weight_maps(first_bucket)
    fb = first_bucket
    if fb == 0:
        summed_map = lambda i: (jnp.maximum(i - 1, 0), 0)
    else:
        summed_map = lambda i: (i, 0)
    in_specs = [
        pl.BlockSpec((ROWS_PER_TILE, D), lambda i: (i + fb, 0)),
        pl.BlockSpec((ROWS_PER_TILE, D), summed_map),
        pl.BlockSpec((1, D, D), lambda i: (self_idx(i), 0, 0)),
        pl.BlockSpec((1, D, D), lambda i: (rel_idx(i), 0, 0)),
        pl.BlockSpec((1, 1, D), lambda i: (self_idx(i), 0, 0)),
        pl.BlockSpec((1, 1, D), lambda i: (rel_idx(i), 0, 0)),
    ]
    args = [atom_features, summed, W, W, b3, b3]
    kw = {}
    if partial_out is None:
        body = functools.partial(_tc_body, first_bucket=fb)
    else:
        body = functools.partial(_tc_body_alias, first_bucket=fb)
        in_specs.append(pl.BlockSpec(memory_space=pl.ANY))
        args.append(partial_out)
        kw["input_output_aliases"] = {6: 0}
    return pl.pallas_call(
        body,
        grid=(n_buckets,),
        in_specs=in_specs,
        out_specs=pl.BlockSpec((ROWS_PER_TILE, D), lambda i: (i + fb, 0)),
        out_shape=jax.ShapeDtypeStruct((N_ATOMS, D), jnp.float32),
        **kw,
    )(*args)


def kernel(atom_features, deg_slice, membership, deg_adj_1, deg_adj_2,
           deg_adj_3, deg_adj_4, deg_adj_5, deg_adj_6, deg_adj_7, deg_adj_8,
           deg_adj_9, deg_adj_10, W, b):
    adjs = [deg_adj_1, deg_adj_2, deg_adj_3, deg_adj_4, deg_adj_5,
            deg_adj_6, deg_adj_7, deg_adj_8, deg_adj_9, deg_adj_10]
    idx_cols = [a.T.reshape(-1) for a in adjs]
    b3 = b.reshape(2 * MAX_DEG + 1, 1, D)
    summed = [_sc_gather_sum(atom_features, idx_cols, degs) for degs in STAGES]
    out = None
    fb = 0
    for si, degs in enumerate(STAGES):
        nb = len(degs) + (1 if si == 0 else 0)
        out = _tc_matmul(atom_features, summed[si], W, b3, fb, nb,
                         partial_out=out)
        fb += nb
    return out
